# baseline (speedup 1.0000x reference)
import functools

import jax
import jax.numpy as jnp
from jax import lax
from jax.experimental import pallas as pl
from jax.experimental.pallas import tpu as pltpu
from jax.experimental.pallas import tpu_sc as plsc


def kernel(labels, table):
    (B,) = labels.shape
    V, D = table.shape
    info = plsc.get_sparse_core_info()
    nw = info.num_cores * info.num_subcores
    b_per_w = B // nw

    mesh = plsc.VectorSubcoreMesh(core_axis_name="c", subcore_axis_name="s")

    @functools.partial(
        pl.kernel,
        mesh=mesh,
        out_type=jax.ShapeDtypeStruct((B, D), jnp.float32),
        scratch_types=[
            pltpu.VMEM((b_per_w,), jnp.int32),
            pltpu.VMEM((b_per_w, D), jnp.float32),
            pltpu.SemaphoreType.DMA,
        ],
    )
    def gather_kernel(labels_hbm, table_hbm, out_hbm, idx_v, out_v, sem):
        wid = lax.axis_index("s") * info.num_cores + lax.axis_index("c")
        base = wid * b_per_w
        pltpu.sync_copy(labels_hbm.at[pl.ds(base, b_per_w)], idx_v)

        table_3d = table_hbm.reshape(V // 8, 8, D)

        def fire_group(g, _):
            v = idx_v[pl.ds(g * 16, 16)]
            for k in range(16):
                l = v[k]
                pltpu.make_async_copy(
                    table_3d.at[lax.shift_right_logical(l, 3), l & 7],
                    out_v.at[g * 16 + k],
                    sem,
                ).start()
            return _

        lax.fori_loop(0, b_per_w // 16, fire_group, 0)

        pltpu.make_async_copy(
            out_hbm.at[pl.ds(base, b_per_w)],
            out_v,
            sem,
        ).wait()

        pltpu.sync_copy(out_v, out_hbm.at[pl.ds(base, b_per_w)])

    return gather_kernel(labels.astype(jnp.int32), table)


# speedup vs baseline: 1.6647x; 1.6647x over previous
import functools

import jax
import jax.numpy as jnp
from jax import lax
from jax.experimental import pallas as pl
from jax.experimental.pallas import tpu as pltpu
from jax.experimental.pallas import tpu_sc as plsc


def kernel(labels, table):
    (B,) = labels.shape
    V, D = table.shape
    info = plsc.get_sparse_core_info()
    nw = info.num_cores * info.num_subcores
    b_per_w = B // nw

    table3 = table.reshape(V // 8, 8, D)

    mesh = plsc.VectorSubcoreMesh(core_axis_name="c", subcore_axis_name="s")

    @functools.partial(
        pl.kernel,
        mesh=mesh,
        out_type=jax.ShapeDtypeStruct((B, D), jnp.float32),
        scratch_types=[
            pltpu.VMEM((b_per_w,), jnp.int32),
            pltpu.VMEM((b_per_w, D), jnp.float32),
            pltpu.SemaphoreType.DMA,
        ],
    )
    def gather_kernel(labels_hbm, table_hbm, out_hbm, idx_v, out_v, sem):
        wid = lax.axis_index("s") * info.num_cores + lax.axis_index("c")
        base = wid * b_per_w
        pltpu.sync_copy(labels_hbm.at[pl.ds(base, b_per_w)], idx_v)

        def fire_group(g, _):
            v = idx_v[pl.ds(g * 16, 16)]
            for k in range(16):
                l = v[k]
                pltpu.make_async_copy(
                    table_hbm.at[lax.shift_right_logical(l, 3), l & 7],
                    out_v.at[g * 16 + k],
                    sem,
                ).start()
            return _

        lax.fori_loop(0, b_per_w // 16, fire_group, 0)

        pltpu.make_async_copy(
            out_hbm.at[pl.ds(base, b_per_w)],
            out_v,
            sem,
        ).wait()

        pltpu.sync_copy(out_v, out_hbm.at[pl.ds(base, b_per_w)])

    return gather_kernel(labels.astype(jnp.int32), table3)
